# Initial kernel scaffold; baseline (speedup 1.0000x reference)
import jax
import jax.numpy as jnp
from jax.experimental import pallas as pl


def kernel(biomarkers, W_enc, b_enc, Wq, Wk, Wv, Wo, bq, bk, bv, bo, W1, b1, W2, b2, g1, bt1, g2, bt2, Wr, br, We1, bm1, We2, bm2):
    raise NotImplementedError("write your pallas kernel here")


# speedup vs baseline: 1.3373x; 1.3373x over previous
import jax
import jax.numpy as jnp
from jax.experimental import pallas as pl

T = 2048
D = 1024
H = 16
DH = 64
FT = 2048
E = 8
FE = 1024
BT = 256


def _bdot(a, b):
    return jnp.dot(a.astype(jnp.bfloat16), b.astype(jnp.bfloat16),
                   preferred_element_type=jnp.float32)


def _linear(x, w, b):
    t, k = x.shape
    n = w.shape[1]

    def kern(x_ref, w_ref, b_ref, o_ref):
        o_ref[...] = (
            _bdot(x_ref[...], w_ref[...])
            + b_ref[...]
        )

    return pl.pallas_call(
        kern,
        grid=(t // BT,),
        in_specs=[
            pl.BlockSpec((BT, k), lambda i: (i, 0)),
            pl.BlockSpec((k, n), lambda i: (0, 0)),
            pl.BlockSpec((1, n), lambda i: (0, 0)),
        ],
        out_specs=pl.BlockSpec((BT, n), lambda i: (i, 0)),
        out_shape=jax.ShapeDtypeStruct((t, n), jnp.float32),
    )(x, w, b.reshape(1, n))


def _attention(q, k, v):
    def kern(q_ref, k_ref, v_ref, o_ref):
        outs = []
        for s in range(2):
            qs = q_ref[...][:, s * DH:(s + 1) * DH]
            ks = k_ref[...][:, s * DH:(s + 1) * DH]
            vs = v_ref[...][:, s * DH:(s + 1) * DH]
            sc = jax.lax.dot_general(
                qs.astype(jnp.bfloat16), ks.astype(jnp.bfloat16),
                (((1,), (1,)), ((), ())),
                preferred_element_type=jnp.float32,
            ) * 0.125
            sc0, sc1 = sc[:, :1024], sc[:, 1024:]
            v0, v1 = vs[:1024], vs[1024:]
            m0 = jnp.max(sc0, axis=1, keepdims=True)
            p0 = jnp.exp(sc0 - m0)
            acc = _bdot(p0, v0)
            l0 = jnp.sum(p0, axis=1, keepdims=True)
            m1 = jnp.maximum(m0, jnp.max(sc1, axis=1, keepdims=True))
            alpha = jnp.exp(m0 - m1)
            p1 = jnp.exp(sc1 - m1)
            acc = acc * alpha + _bdot(p1, v1)
            li = l0 * alpha + jnp.sum(p1, axis=1, keepdims=True)
            outs.append(acc / li)
        o_ref[...] = jnp.concatenate(outs, axis=1)

    return pl.pallas_call(
        kern,
        grid=(H // 2, T // BT),
        in_specs=[
            pl.BlockSpec((BT, 128), lambda p, i: (i, p)),
            pl.BlockSpec((T, 128), lambda p, i: (0, p)),
            pl.BlockSpec((T, 128), lambda p, i: (0, p)),
        ],
        out_specs=pl.BlockSpec((BT, 128), lambda p, i: (i, p)),
        out_shape=jax.ShapeDtypeStruct((T, D), jnp.float32),
    )(q, k, v)


def _oproj_ln(a, w, b, x, g, beta):
    def kern(a_ref, w_ref, b_ref, x_ref, g_ref, bt_ref, o_ref):
        y = (
            x_ref[...]
            + _bdot(a_ref[...], w_ref[...])
            + b_ref[...]
        )
        m = jnp.mean(y, axis=1, keepdims=True)
        var = jnp.mean((y - m) ** 2, axis=1, keepdims=True)
        o_ref[...] = (y - m) / jnp.sqrt(var + 1e-5) * g_ref[...] + bt_ref[...]

    return pl.pallas_call(
        kern,
        grid=(T // BT,),
        in_specs=[
            pl.BlockSpec((BT, D), lambda i: (i, 0)),
            pl.BlockSpec((D, D), lambda i: (0, 0)),
            pl.BlockSpec((1, D), lambda i: (0, 0)),
            pl.BlockSpec((BT, D), lambda i: (i, 0)),
            pl.BlockSpec((1, D), lambda i: (0, 0)),
            pl.BlockSpec((1, D), lambda i: (0, 0)),
        ],
        out_specs=pl.BlockSpec((BT, D), lambda i: (i, 0)),
        out_shape=jax.ShapeDtypeStruct((T, D), jnp.float32),
    )(a, w, b.reshape(1, D), x, g.reshape(1, D), beta.reshape(1, D))


def _ffn_ln(x, w1, b1, w2, b2, g, beta):
    def kern(x_ref, w1_ref, b1_ref, w2_ref, b2_ref, g_ref, bt_ref, o_ref):
        xx = x_ref[...]
        h = jnp.maximum(
            _bdot(xx, w1_ref[...])
            + b1_ref[...],
            0.0,
        )
        y = (
            xx
            + _bdot(h, w2_ref[...])
            + b2_ref[...]
        )
        m = jnp.mean(y, axis=1, keepdims=True)
        var = jnp.mean((y - m) ** 2, axis=1, keepdims=True)
        o_ref[...] = (y - m) / jnp.sqrt(var + 1e-5) * g_ref[...] + bt_ref[...]

    return pl.pallas_call(
        kern,
        grid=(T // BT,),
        in_specs=[
            pl.BlockSpec((BT, D), lambda i: (i, 0)),
            pl.BlockSpec((D, FT), lambda i: (0, 0)),
            pl.BlockSpec((1, FT), lambda i: (0, 0)),
            pl.BlockSpec((FT, D), lambda i: (0, 0)),
            pl.BlockSpec((1, D), lambda i: (0, 0)),
            pl.BlockSpec((1, D), lambda i: (0, 0)),
            pl.BlockSpec((1, D), lambda i: (0, 0)),
        ],
        out_specs=pl.BlockSpec((BT, D), lambda i: (i, 0)),
        out_shape=jax.ShapeDtypeStruct((T, D), jnp.float32),
    )(x, w1, b1.reshape(1, FT), w2, b2.reshape(1, D), g.reshape(1, D), beta.reshape(1, D))


def _router(x, wr_pad, br_pad):
    def kern(x_ref, wr_ref, br_ref, g_ref):
        l = (
            _bdot(x_ref[...], wr_ref[...])
            + br_ref[...]
        )
        idx = jax.lax.broadcasted_iota(jnp.int32, l.shape, 1)
        m1 = jnp.max(l, axis=1, keepdims=True)
        i1 = jnp.min(jnp.where(l >= m1, idx, 127), axis=1, keepdims=True)
        first1 = idx == i1
        l2 = jnp.where(first1, -1e30, l)
        m2 = jnp.max(l2, axis=1, keepdims=True)
        i2 = jnp.min(jnp.where(l2 >= m2, idx, 127), axis=1, keepdims=True)
        first2 = idx == i2
        e2 = jnp.exp(m2 - m1)
        ga = 1.0 / (1.0 + e2)
        gb = e2 / (1.0 + e2)
        g_ref[...] = (
            first1.astype(jnp.float32) * ga + first2.astype(jnp.float32) * gb
        )

    return pl.pallas_call(
        kern,
        grid=(T // BT,),
        in_specs=[
            pl.BlockSpec((BT, D), lambda i: (i, 0)),
            pl.BlockSpec((D, 128), lambda i: (0, 0)),
            pl.BlockSpec((1, 128), lambda i: (0, 0)),
        ],
        out_specs=pl.BlockSpec((BT, 128), lambda i: (i, 0)),
        out_shape=jax.ShapeDtypeStruct((T, 128), jnp.float32),
    )(x, wr_pad, br_pad)


def _moe_dense(x, gm, we1, bm1, we2, bm2):
    def kern(x_ref, g_ref, w1_ref, b1_ref, w2_ref, b2_ref, o_ref):
        e = pl.program_id(1)
        xx = x_ref[...]
        h = jnp.maximum(
            _bdot(xx, w1_ref[0])
            + b1_ref[0],
            0.0,
        )
        y = _bdot(h, w2_ref[0]) + b2_ref[0]
        gcol = g_ref[...]
        onehot = (
            jax.lax.broadcasted_iota(jnp.int32, gcol.shape, 1) == e
        ).astype(jnp.float32)
        ge = jnp.sum(gcol * onehot, axis=1, keepdims=True)
        contrib = y * ge

        @pl.when(e == 0)
        def _():
            o_ref[...] = contrib

        @pl.when(e > 0)
        def _():
            o_ref[...] = o_ref[...] + contrib

    return pl.pallas_call(
        kern,
        grid=(T // BT, E),
        in_specs=[
            pl.BlockSpec((BT, D), lambda j, e: (j, 0)),
            pl.BlockSpec((BT, 128), lambda j, e: (j, 0)),
            pl.BlockSpec((1, D, FE), lambda j, e: (e, 0, 0)),
            pl.BlockSpec((1, 1, FE), lambda j, e: (e, 0, 0)),
            pl.BlockSpec((1, FE, D), lambda j, e: (e, 0, 0)),
            pl.BlockSpec((1, 1, D), lambda j, e: (e, 0, 0)),
        ],
        out_specs=pl.BlockSpec((BT, D), lambda j, e: (j, 0)),
        out_shape=jax.ShapeDtypeStruct((T, D), jnp.float32),
    )(x, gm, we1, bm1.reshape(E, 1, FE), we2, bm2.reshape(E, 1, D))


def kernel(biomarkers, W_enc, b_enc, Wq, Wk, Wv, Wo, bq, bk, bv, bo, W1, b1,
           W2, b2, g1, bt1, g2, bt2, Wr, br, We1, bm1, We2, bm2):
    x = _linear(biomarkers, W_enc, b_enc)
    for l in range(2):
        wqkv = jnp.concatenate([Wq[l], Wk[l], Wv[l]], axis=1)
        bqkv = jnp.concatenate([bq[l], bk[l], bv[l]])
        qkv = _linear(x, wqkv, bqkv)
        q, k, v = qkv[:, :D], qkv[:, D:2 * D], qkv[:, 2 * D:]
        a = _attention(q, k, v)
        x = _oproj_ln(a, Wo[l], bo[l], x, g1[l], bt1[l])
        x = _ffn_ln(x, W1[l], b1[l], W2[l], b2[l], g2[l], bt2[l])
    wr_pad = jnp.zeros((D, 128), jnp.float32).at[:, :E].set(Wr)
    br_pad = jnp.full((1, 128), -1e30, jnp.float32).at[0, :E].set(br)
    gm = _router(x, wr_pad, br_pad)
    return _moe_dense(x, gm, We1, bm1, We2, bm2)
